# baseline (speedup 1.0000x reference)
import functools

import jax
import jax.numpy as jnp
from jax import lax
from jax.experimental import pallas as pl
from jax.experimental.pallas import tpu as pltpu
from jax.experimental.pallas import tpu_sc as plsc

N = 100000
Q = 1024
D = 128
BLK = 4000
T = N // BLK
EPS = 1e-8
BIG = 2**30

_SC_INFO = plsc.get_sparse_core_info()
_NC = _SC_INFO.num_cores
_NS = _SC_INFO.num_subcores
_NW = _NC * _NS
_QW = Q // _NW
_LANES = 16


def _stage(s, w_ref, znt_ref, wn_w, wn_r, sc_w, sc_r, pm_acc, pi_acc):
    w = w_ref[...]
    wnorm = jnp.maximum(jnp.sqrt(jnp.sum(w * w, axis=1, keepdims=True)), EPS)
    wn_w[...] = w / wnorm

    sc_w[...] = lax.dot_general(
        wn_r[...], znt_ref[...],
        (((1,), (0,)), ((), ())),
        preferred_element_type=jnp.float32,
        precision=lax.Precision.DEFAULT,
    )

    t = jnp.maximum(s - 2, 0)
    scores3 = sc_r[...].reshape(BLK // 8, 8, Q)
    run = scores3[0]
    ridx = jnp.zeros((8, Q), jnp.int32)
    for r in range(1, BLK // 8):
        sv = scores3[r]
        gt = sv > run
        run = jnp.maximum(run, sv)
        ridx = jnp.where(gt, r, ridx)
    rid = ridx * 8 + lax.broadcasted_iota(jnp.int32, (8, Q), 0)
    m = jnp.max(run, axis=0, keepdims=True)
    cand = jnp.min(jnp.where(run == m, rid, BIG), axis=0, keepdims=True) + t * BLK
    pm_acc[pl.ds(t, 1), :] = m
    pi_acc[pl.ds(t, 1), :] = cand


def _tc_body(z_ref, w_ref, pmax_ref, pidx_ref,
             znt_ref, wn_a, wn_b, sc_a, sc_b, pm_acc, pi_acc):
    s = pl.program_id(0)

    @pl.when(s == 0)
    def _init():
        z = z_ref[...]
        znorm = jnp.maximum(jnp.sqrt(jnp.sum(z * z, axis=1, keepdims=True)), EPS)
        znt_ref[...] = (z / znorm).T

    @pl.when(s % 2 == 0)
    def _even():
        _stage(s, w_ref, znt_ref, wn_a, wn_b, sc_a, sc_b, pm_acc, pi_acc)

    @pl.when(s % 2 == 1)
    def _odd():
        _stage(s, w_ref, znt_ref, wn_b, wn_a, sc_b, sc_a, pm_acc, pi_acc)

    @pl.when(s == T + 1)
    def _flush():
        pmax_ref[...] = pm_acc[...]
        pidx_ref[...] = pi_acc[...]


def _tc_partials(z, embedding_weight):
    return pl.pallas_call(
        _tc_body,
        grid=(T + 2,),
        in_specs=[
            pl.BlockSpec((Q, D), lambda s: (0, 0)),
            pl.BlockSpec((BLK, D), lambda s: (jnp.minimum(s, T - 1), 0)),
        ],
        out_specs=[
            pl.BlockSpec((T, Q), lambda s: (0, 0)),
            pl.BlockSpec((T, Q), lambda s: (0, 0)),
        ],
        out_shape=[
            jax.ShapeDtypeStruct((T, Q), jnp.float32),
            jax.ShapeDtypeStruct((T, Q), jnp.int32),
        ],
        scratch_shapes=[
            pltpu.VMEM((D, Q), jnp.float32),
            pltpu.VMEM((BLK, D), jnp.float32),
            pltpu.VMEM((BLK, D), jnp.float32),
            pltpu.VMEM((BLK, Q), jnp.float32),
            pltpu.VMEM((BLK, Q), jnp.float32),
            pltpu.VMEM((T, Q), jnp.float32),
            pltpu.VMEM((T, Q), jnp.int32),
        ],
    )(z, embedding_weight)


@functools.partial(
    pl.kernel,
    out_type=jax.ShapeDtypeStruct((Q,), jnp.int32),
    mesh=plsc.VectorSubcoreMesh(core_axis_name="c", subcore_axis_name="s"),
    scratch_types=[
        pltpu.VMEM((T, 128), jnp.float32),
        pltpu.VMEM((T, 128), jnp.int32),
        pltpu.VMEM((_QW,), jnp.int32),
    ],
)
def _sc_merge(pmax_hbm, pidx_hbm, out_hbm, vmax, vidx, vout):
    wid = lax.axis_index("s") * _NC + lax.axis_index("c")
    base = wid * _QW
    blk_base = (wid // 4) * 128
    loc = (wid % 4) * _QW
    pltpu.sync_copy(pmax_hbm.at[:, pl.ds(blk_base, 128)], vmax)
    pltpu.sync_copy(pidx_hbm.at[:, pl.ds(blk_base, 128)], vidx)
    for g in range(_QW // _LANES):
        col = loc + g * _LANES

        def body(t, carry):
            run, widx = carry
            pv = vmax[t, pl.ds(col, _LANES)]
            iv = vidx[t, pl.ds(col, _LANES)]
            gt = pv > run
            run = jnp.where(gt, pv, run)
            widx = jnp.where(gt, iv, widx)
            return run, widx

        run0 = vmax[0, pl.ds(col, _LANES)]
        widx0 = vidx[0, pl.ds(col, _LANES)]
        _, widx = lax.fori_loop(1, T, body, (run0, widx0))
        vout[pl.ds(g * _LANES, _LANES)] = widx
    pltpu.sync_copy(vout, out_hbm.at[pl.ds(base, _QW)])


def kernel(z, embedding_weight):
    pmax, pidx = _tc_partials(z, embedding_weight)
    return _sc_merge(pmax, pidx)


# speedup vs baseline: 1.4793x; 1.4793x over previous
import functools

import jax
import jax.numpy as jnp
from jax import lax
from jax.experimental import pallas as pl
from jax.experimental.pallas import tpu as pltpu
from jax.experimental.pallas import tpu_sc as plsc

N = 100000
Q = 1024
D = 128
BLK = 4000
T = N // BLK
EPS = 1e-8
BIG = 2**30

_SC_INFO = plsc.get_sparse_core_info()
_NC = _SC_INFO.num_cores
_NS = _SC_INFO.num_subcores
_NW = _NC * _NS
_QW = Q // _NW
_LANES = 16


def _tc_body(z_ref, w_ref, pmax_ref, pidx_ref, znt_ref, pm_acc, pi_acc):
    i = pl.program_id(0)

    @pl.when(i == 0)
    def _init():
        z = z_ref[...]
        znorm = jnp.maximum(jnp.sqrt(jnp.sum(z * z, axis=1, keepdims=True)), EPS)
        znt_ref[...] = (z / znorm).T

    w = w_ref[...]
    wnorm = jnp.maximum(jnp.sqrt(jnp.sum(w * w, axis=1, keepdims=True)), EPS)
    wn = w / wnorm
    scores = lax.dot_general(
        wn, znt_ref[...],
        (((1,), (0,)), ((), ())),
        preferred_element_type=jnp.float32,
        precision=lax.Precision.DEFAULT,
    )

    scores3 = scores.reshape(BLK // 8, 8, Q)
    run = scores3[0]
    ridx = jnp.zeros((8, Q), jnp.int32)
    for r in range(1, BLK // 8):
        sv = scores3[r]
        gt = sv > run
        run = jnp.maximum(run, sv)
        ridx = jnp.where(gt, r, ridx)
    rid = ridx * 8 + lax.broadcasted_iota(jnp.int32, (8, Q), 0)
    m = jnp.max(run, axis=0, keepdims=True)
    cand = jnp.min(jnp.where(run == m, rid, BIG), axis=0, keepdims=True) + i * BLK
    pm_acc[pl.ds(i, 1), :] = m
    pi_acc[pl.ds(i, 1), :] = cand

    @pl.when(i == T - 1)
    def _flush():
        pmax_ref[...] = pm_acc[...]
        pidx_ref[...] = pi_acc[...]


def _tc_partials(z, embedding_weight):
    return pl.pallas_call(
        _tc_body,
        grid=(T,),
        in_specs=[
            pl.BlockSpec((Q, D), lambda i: (0, 0)),
            pl.BlockSpec((BLK, D), lambda i: (i, 0)),
        ],
        out_specs=[
            pl.BlockSpec((T, Q), lambda s: (0, 0)),
            pl.BlockSpec((T, Q), lambda s: (0, 0)),
        ],
        out_shape=[
            jax.ShapeDtypeStruct((T, Q), jnp.float32),
            jax.ShapeDtypeStruct((T, Q), jnp.int32),
        ],
        scratch_shapes=[
            pltpu.VMEM((D, Q), jnp.float32),
            pltpu.VMEM((T, Q), jnp.float32),
            pltpu.VMEM((T, Q), jnp.int32),
        ],
    )(z, embedding_weight)


@functools.partial(
    pl.kernel,
    out_type=jax.ShapeDtypeStruct((Q,), jnp.int32),
    mesh=plsc.VectorSubcoreMesh(core_axis_name="c", subcore_axis_name="s"),
    scratch_types=[
        pltpu.VMEM((T, 128), jnp.float32),
        pltpu.VMEM((T, 128), jnp.int32),
        pltpu.VMEM((_QW,), jnp.int32),
    ],
)
def _sc_merge(pmax_hbm, pidx_hbm, out_hbm, vmax, vidx, vout):
    wid = lax.axis_index("s") * _NC + lax.axis_index("c")
    base = wid * _QW
    blk_base = (wid // 4) * 128
    loc = (wid % 4) * _QW
    pltpu.sync_copy(pmax_hbm.at[:, pl.ds(blk_base, 128)], vmax)
    pltpu.sync_copy(pidx_hbm.at[:, pl.ds(blk_base, 128)], vidx)
    for g in range(_QW // _LANES):
        col = loc + g * _LANES

        def body(t, carry):
            run, widx = carry
            pv = vmax[t, pl.ds(col, _LANES)]
            iv = vidx[t, pl.ds(col, _LANES)]
            gt = pv > run
            run = jnp.where(gt, pv, run)
            widx = jnp.where(gt, iv, widx)
            return run, widx

        run0 = vmax[0, pl.ds(col, _LANES)]
        widx0 = vidx[0, pl.ds(col, _LANES)]
        _, widx = lax.fori_loop(1, T, body, (run0, widx0))
        vout[pl.ds(g * _LANES, _LANES)] = widx
    pltpu.sync_copy(vout, out_hbm.at[pl.ds(base, _QW)])


def kernel(z, embedding_weight):
    pmax, pidx = _tc_partials(z, embedding_weight)
    return _sc_merge(pmax, pidx)


# speedup vs baseline: 1.5014x; 1.0149x over previous
import functools

import jax
import jax.numpy as jnp
from jax import lax
from jax.experimental import pallas as pl
from jax.experimental.pallas import tpu as pltpu
from jax.experimental.pallas import tpu_sc as plsc

N = 100000
Q = 1024
D = 128
BLK = 5000
T = N // BLK
EPS = 1e-8
BIG = 2**30

_SC_INFO = plsc.get_sparse_core_info()
_NC = _SC_INFO.num_cores
_NS = _SC_INFO.num_subcores
_NW = _NC * _NS
_QW = Q // _NW
_LANES = 16


def _tc_body(z_ref, w_ref, pmax_ref, pidx_ref, znt_ref, pm_acc, pi_acc):
    i = pl.program_id(0)

    @pl.when(i == 0)
    def _init():
        z = z_ref[...]
        znorm = jnp.maximum(jnp.sqrt(jnp.sum(z * z, axis=1, keepdims=True)), EPS)
        znt_ref[...] = (z / znorm).T

    w = w_ref[...]
    wnorm = jnp.maximum(jnp.sqrt(jnp.sum(w * w, axis=1, keepdims=True)), EPS)
    wn = w / wnorm
    scores = lax.dot_general(
        wn, znt_ref[...],
        (((1,), (0,)), ((), ())),
        preferred_element_type=jnp.float32,
        precision=lax.Precision.DEFAULT,
    )

    scores3 = scores.reshape(BLK // 8, 8, Q)
    run = scores3[0]
    ridx = jnp.zeros((8, Q), jnp.int32)
    for r in range(1, BLK // 8):
        sv = scores3[r]
        gt = sv > run
        run = jnp.maximum(run, sv)
        ridx = jnp.where(gt, r, ridx)
    rid = ridx * 8 + lax.broadcasted_iota(jnp.int32, (8, Q), 0)
    m = jnp.max(run, axis=0, keepdims=True)
    cand = jnp.min(jnp.where(run == m, rid, BIG), axis=0, keepdims=True) + i * BLK
    pm_acc[pl.ds(i, 1), :] = m
    pi_acc[pl.ds(i, 1), :] = cand

    @pl.when(i == T - 1)
    def _flush():
        pmax_ref[...] = pm_acc[...]
        pidx_ref[...] = pi_acc[...]


def _tc_partials(z, embedding_weight):
    return pl.pallas_call(
        _tc_body,
        grid=(T,),
        in_specs=[
            pl.BlockSpec((Q, D), lambda i: (0, 0)),
            pl.BlockSpec((BLK, D), lambda i: (i, 0)),
        ],
        out_specs=[
            pl.BlockSpec((T, Q), lambda s: (0, 0)),
            pl.BlockSpec((T, Q), lambda s: (0, 0)),
        ],
        out_shape=[
            jax.ShapeDtypeStruct((T, Q), jnp.float32),
            jax.ShapeDtypeStruct((T, Q), jnp.int32),
        ],
        scratch_shapes=[
            pltpu.VMEM((D, Q), jnp.float32),
            pltpu.VMEM((T, Q), jnp.float32),
            pltpu.VMEM((T, Q), jnp.int32),
        ],
    )(z, embedding_weight)


@functools.partial(
    pl.kernel,
    out_type=jax.ShapeDtypeStruct((Q,), jnp.int32),
    mesh=plsc.VectorSubcoreMesh(core_axis_name="c", subcore_axis_name="s"),
    scratch_types=[
        pltpu.VMEM((T, 128), jnp.float32),
        pltpu.VMEM((T, 128), jnp.int32),
        pltpu.VMEM((_QW,), jnp.int32),
    ],
)
def _sc_merge(pmax_hbm, pidx_hbm, out_hbm, vmax, vidx, vout):
    wid = lax.axis_index("s") * _NC + lax.axis_index("c")
    base = wid * _QW
    blk_base = (wid // 4) * 128
    loc = (wid % 4) * _QW
    pltpu.sync_copy(pmax_hbm.at[:, pl.ds(blk_base, 128)], vmax)
    pltpu.sync_copy(pidx_hbm.at[:, pl.ds(blk_base, 128)], vidx)
    for g in range(_QW // _LANES):
        col = loc + g * _LANES

        def body(t, carry):
            run, widx = carry
            pv = vmax[t, pl.ds(col, _LANES)]
            iv = vidx[t, pl.ds(col, _LANES)]
            gt = pv > run
            run = jnp.where(gt, pv, run)
            widx = jnp.where(gt, iv, widx)
            return run, widx

        run0 = vmax[0, pl.ds(col, _LANES)]
        widx0 = vidx[0, pl.ds(col, _LANES)]
        _, widx = lax.fori_loop(1, T, body, (run0, widx0))
        vout[pl.ds(g * _LANES, _LANES)] = widx
    pltpu.sync_copy(vout, out_hbm.at[pl.ds(base, _QW)])


def kernel(z, embedding_weight):
    pmax, pidx = _tc_partials(z, embedding_weight)
    return _sc_merge(pmax, pidx)


# speedup vs baseline: 1.5316x; 1.0201x over previous
import functools

import jax
import jax.numpy as jnp
from jax import lax
from jax.experimental import pallas as pl
from jax.experimental.pallas import tpu as pltpu
from jax.experimental.pallas import tpu_sc as plsc

N = 100000
Q = 1024
D = 128
BLK = 10000
T = N // BLK
EPS = 1e-8
BIG = 2**30

_SC_INFO = plsc.get_sparse_core_info()
_NC = _SC_INFO.num_cores
_NS = _SC_INFO.num_subcores
_NW = _NC * _NS
_QW = Q // _NW
_LANES = 16


def _tc_body(z_ref, w_ref, pmax_ref, pidx_ref, znt_ref, pm_acc, pi_acc):
    i = pl.program_id(0)

    @pl.when(i == 0)
    def _init():
        z = z_ref[...]
        znorm = jnp.maximum(jnp.sqrt(jnp.sum(z * z, axis=1, keepdims=True)), EPS)
        znt_ref[...] = (z / znorm).T

    w = w_ref[...]
    wnorm = jnp.maximum(jnp.sqrt(jnp.sum(w * w, axis=1, keepdims=True)), EPS)
    wn = w / wnorm
    scores = lax.dot_general(
        wn, znt_ref[...],
        (((1,), (0,)), ((), ())),
        preferred_element_type=jnp.float32,
        precision=lax.Precision.DEFAULT,
    )

    scores3 = scores.reshape(BLK // 8, 8, Q)
    run = scores3[0]
    ridx = jnp.zeros((8, Q), jnp.int32)
    for r in range(1, BLK // 8):
        sv = scores3[r]
        gt = sv > run
        run = jnp.maximum(run, sv)
        ridx = jnp.where(gt, r, ridx)
    rid = ridx * 8 + lax.broadcasted_iota(jnp.int32, (8, Q), 0)
    m = jnp.max(run, axis=0, keepdims=True)
    cand = jnp.min(jnp.where(run == m, rid, BIG), axis=0, keepdims=True) + i * BLK
    pm_acc[pl.ds(i, 1), :] = m
    pi_acc[pl.ds(i, 1), :] = cand

    @pl.when(i == T - 1)
    def _flush():
        pmax_ref[...] = pm_acc[...]
        pidx_ref[...] = pi_acc[...]


def _tc_partials(z, embedding_weight):
    return pl.pallas_call(
        _tc_body,
        grid=(T,),
        in_specs=[
            pl.BlockSpec((Q, D), lambda i: (0, 0)),
            pl.BlockSpec((BLK, D), lambda i: (i, 0)),
        ],
        out_specs=[
            pl.BlockSpec((T, Q), lambda s: (0, 0)),
            pl.BlockSpec((T, Q), lambda s: (0, 0)),
        ],
        out_shape=[
            jax.ShapeDtypeStruct((T, Q), jnp.float32),
            jax.ShapeDtypeStruct((T, Q), jnp.int32),
        ],
        scratch_shapes=[
            pltpu.VMEM((D, Q), jnp.float32),
            pltpu.VMEM((T, Q), jnp.float32),
            pltpu.VMEM((T, Q), jnp.int32),
        ],
    )(z, embedding_weight)


@functools.partial(
    pl.kernel,
    out_type=jax.ShapeDtypeStruct((Q,), jnp.int32),
    mesh=plsc.VectorSubcoreMesh(core_axis_name="c", subcore_axis_name="s"),
    scratch_types=[
        pltpu.VMEM((T, 128), jnp.float32),
        pltpu.VMEM((T, 128), jnp.int32),
        pltpu.VMEM((_QW,), jnp.int32),
    ],
)
def _sc_merge(pmax_hbm, pidx_hbm, out_hbm, vmax, vidx, vout):
    wid = lax.axis_index("s") * _NC + lax.axis_index("c")
    base = wid * _QW
    blk_base = (wid // 4) * 128
    loc = (wid % 4) * _QW
    pltpu.sync_copy(pmax_hbm.at[:, pl.ds(blk_base, 128)], vmax)
    pltpu.sync_copy(pidx_hbm.at[:, pl.ds(blk_base, 128)], vidx)
    for g in range(_QW // _LANES):
        col = loc + g * _LANES

        def body(t, carry):
            run, widx = carry
            pv = vmax[t, pl.ds(col, _LANES)]
            iv = vidx[t, pl.ds(col, _LANES)]
            gt = pv > run
            run = jnp.where(gt, pv, run)
            widx = jnp.where(gt, iv, widx)
            return run, widx

        run0 = vmax[0, pl.ds(col, _LANES)]
        widx0 = vidx[0, pl.ds(col, _LANES)]
        _, widx = lax.fori_loop(1, T, body, (run0, widx0))
        vout[pl.ds(g * _LANES, _LANES)] = widx
    pltpu.sync_copy(vout, out_hbm.at[pl.ds(base, _QW)])


def kernel(z, embedding_weight):
    pmax, pidx = _tc_partials(z, embedding_weight)
    return _sc_merge(pmax, pidx)
